# baseline (speedup 1.0000x reference)
import functools

import jax
import jax.numpy as jnp
from jax import lax
from jax.experimental import pallas as pl
from jax.experimental.pallas import tpu as pltpu
from jax.experimental.pallas import tpu_sc as plsc

B = 16384
F = 26
FEAT_NUM = 100000
K = 16
FEATURE_LENGTH = F * FEAT_NUM

NC = 2
NS = 16
NW = NC * NS
RPW = B // NW
CH = 64
NCH = RPW // CH
CF = CH * F

_mesh = plsc.VectorSubcoreMesh(core_axis_name="c", subcore_axis_name="s")


@functools.partial(
    pl.kernel,
    out_type=jax.ShapeDtypeStruct((B,), jnp.float32),
    mesh=_mesh,
    compiler_params=pltpu.CompilerParams(
        needs_layout_passes=False, use_tc_tiling_on_sc=False),
    scratch_types=[
        pltpu.VMEM((RPW * F,), jnp.int32),
        pltpu.VMEM((CF, K), jnp.float32),
        pltpu.VMEM((CF,), jnp.float32),
        pltpu.VMEM((RPW,), jnp.float32),
        pltpu.SemaphoreType.DMA,
        pltpu.SemaphoreType.DMA,
    ],
)
def _fm_sc(idx_hbm, w_hbm, v_hbm, out_hbm, idx_v, vrows, wrows, out_v,
           semv, semw):
    wid = lax.axis_index("s") * NC + lax.axis_index("c")
    base = wid * RPW

    pltpu.sync_copy(idx_hbm.at[pl.ds(base * F, RPW * F)], idx_v)

    iota = lax.iota(jnp.int32, 16)
    ksplats = [jnp.full((16,), k, jnp.int32) for k in range(K)]
    zero = jnp.zeros((16,), jnp.float32)

    @pl.loop(0, NCH)
    def _chunk(ch):
        idx_sl = idx_v.at[pl.ds(ch * CF, CF)]
        cpv = pltpu.async_copy(v_hbm.at[idx_sl], vrows, semv)
        cpw = pltpu.async_copy(w_hbm.at[idx_sl], wrows, semw)
        cpv.wait()
        cpw.wait()

        @pl.loop(0, CH // 16)
        def _group(g):
            riota_f = (g * 16 + iota) * F
            fidx = [riota_f + f for f in range(F)]

            wacc = zero
            for f in range(F):
                wacc = wacc + plsc.load_gather(wrows, [fidx[f]])

            t2 = zero
            tot = zero
            for k in range(K):
                acc = zero
                for f in range(F):
                    v = plsc.load_gather(vrows, [fidx[f], ksplats[k]])
                    acc = acc + v
                    t2 = t2 + v * v
                tot = tot + acc * acc

            res = wacc + 0.5 * (tot - t2)
            out_v[pl.ds(ch * CH + g * 16, 16)] = res

    pltpu.sync_copy(out_v, out_hbm.at[pl.ds(base, RPW)])


def kernel(inputs, w0, w, V):
    offsets = (jnp.arange(F, dtype=jnp.int32) * FEAT_NUM)[None, :]
    idx = (inputs.astype(jnp.int32) + offsets).reshape(-1)
    out = _fm_sc(idx, w.reshape(-1), V)
    return out[:, None] + w0
